# Initial kernel scaffold; baseline (speedup 1.0000x reference)
import jax
import jax.numpy as jnp
from jax.experimental import pallas as pl


def kernel(x, edge_index, batch, W1, b1, W2, b2, W3, b3, Wl, bl):
    raise NotImplementedError("write your pallas kernel here")


# speedup vs baseline: 24.0899x; 24.0899x over previous
import functools

import jax
import jax.numpy as jnp
from jax import lax
from jax.experimental import pallas as pl
from jax.experimental.pallas import tpu as pltpu
from jax.experimental.pallas import tpu_sc as plsc

_N = 10000
_E = 320000
_C = 16
_G = 64

_NPAD = 10240
_DUMP = _N
_NCORES = 2
_NSUB = 16
_NTILES = _NCORES * _NSUB
_CHUNK = 128
_CPT = 80
_EPT = _CPT * _CHUNK
_EPAD = _NTILES * _EPT
_RPT = _NPAD // _NSUB

_f32 = jnp.float32
_i32 = jnp.int32

_MESH = plsc.VectorSubcoreMesh(core_axis_name="c", subcore_axis_name="s")


def _deg_body(dst_ref, out_ref, acc, idx, ones_v, zeros_v):
    core = lax.axis_index("c")
    sub = lax.axis_index("s")
    w = core * _NSUB + sub
    for i in range(_RPT // 16):
        zeros_v[pl.ds(i * 16, 16)] = jnp.zeros((16,), _f32)
    for i in range(_CHUNK // 16):
        ones_v[pl.ds(i * 16, 16)] = jnp.ones((16,), _f32)
    pltpu.sync_copy(zeros_v, acc.at[pl.ds(sub * _RPT, _RPT)])
    plsc.subcore_barrier()
    pltpu.sync_copy(dst_ref.at[pl.ds(w * _CPT, _CPT), :], idx)

    def body(j, carry):
        pltpu.sync_copy(ones_v, acc.at[idx.at[j]], add=True)
        return carry

    lax.fori_loop(0, _CPT, body, 0)
    plsc.subcore_barrier()
    pltpu.sync_copy(acc.at[pl.ds(sub * _RPT, _RPT)],
                    out_ref.at[pl.ds(core * _NPAD + sub * _RPT, _RPT)])


_deg_call = functools.partial(
    pl.kernel,
    out_type=jax.ShapeDtypeStruct((_NCORES * _NPAD,), _f32),
    mesh=_MESH,
    scratch_types=[
        pltpu.VMEM_SHARED((_NPAD,), _f32),
        pltpu.VMEM((_CPT, _CHUNK), _i32),
        pltpu.VMEM((_CHUNK,), _f32),
        pltpu.VMEM((_RPT,), _f32),
    ],
)(_deg_body)


def _layer_body(s_ref, src_ref, dst_ref, out_ref, acc, sidx, didx, gbuf,
                zbuf, sem):
    core = lax.axis_index("c")
    sub = lax.axis_index("s")
    w = core * _NSUB + sub
    for i in range(_CHUNK):
        zbuf[i, :] = jnp.zeros((16,), _f32)
    for b in range(_RPT // _CHUNK):
        pltpu.sync_copy(zbuf, acc.at[pl.ds(sub * _RPT + b * _CHUNK, _CHUNK), :])
    plsc.subcore_barrier()
    pltpu.sync_copy(src_ref.at[pl.ds(w * _CPT, _CPT), :], sidx)
    pltpu.sync_copy(dst_ref.at[pl.ds(w * _CPT, _CPT), :], didx)

    def body(j, carry):
        pltpu.async_copy(s_ref.at[sidx.at[j]], gbuf, sem).wait()
        pltpu.sync_copy(gbuf, acc.at[didx.at[j]], add=True)
        return carry

    lax.fori_loop(0, _CPT, body, 0)
    plsc.subcore_barrier()
    pltpu.sync_copy(acc.at[pl.ds(sub * _RPT, _RPT), :],
                    out_ref.at[pl.ds(core * _NPAD + sub * _RPT, _RPT), :])


_layer_call = functools.partial(
    pl.kernel,
    out_type=jax.ShapeDtypeStruct((_NCORES * _NPAD, _C), _f32),
    mesh=_MESH,
    compiler_params=pltpu.CompilerParams(use_tc_tiling_on_sc=False),
    scratch_types=[
        pltpu.VMEM_SHARED((_NPAD, _C), _f32),
        pltpu.VMEM((_CPT, _CHUNK), _i32),
        pltpu.VMEM((_CPT, _CHUNK), _i32),
        pltpu.VMEM((_CHUNK, _C), _f32),
        pltpu.VMEM((_CHUNK, _C), _f32),
        pltpu.SemaphoreType.DMA,
    ],
)(_layer_body)


def _prep_body(x_ref, w1_ref, w2_ref, w3_ref, wl_ref, b1_ref, b2_ref,
               b3_ref, t0_ref, cmat_ref):
    f = _f32
    w3l = jnp.dot(w3_ref[...], wl_ref[...], preferred_element_type=f, precision=lax.Precision.HIGHEST)
    w23l = jnp.dot(w2_ref[...], w3l, preferred_element_type=f, precision=lax.Precision.HIGHEST)
    wc = jnp.dot(w1_ref[...], w23l, preferred_element_type=f, precision=lax.Precision.HIGHEST)
    t0_ref[...] = jnp.dot(x_ref[...], wc, preferred_element_type=f, precision=lax.Precision.HIGHEST)
    c1 = jnp.dot(b1_ref[...], w23l, preferred_element_type=f, precision=lax.Precision.HIGHEST)
    c2 = jnp.dot(b2_ref[...], w3l, preferred_element_type=f, precision=lax.Precision.HIGHEST)
    c3 = jnp.dot(b3_ref[...], wl_ref[...], preferred_element_type=f, precision=lax.Precision.HIGHEST)
    cmat_ref[...] = jnp.concatenate(
        [c1, c2, c3, jnp.zeros((5, _C), f)], axis=0)


_prep_call = pl.pallas_call(
    _prep_body,
    out_shape=(jax.ShapeDtypeStruct((_NPAD, _C), _f32),
               jax.ShapeDtypeStruct((8, _C), _f32)),
)


def _scale_body(p_ref, t0_ref, s0_ref, dinv_ref, dinv2_ref):
    deg = p_ref[0] + p_ref[1] + 1.0
    dinv = 1.0 / jnp.sqrt(deg)
    dinv_ref[...] = dinv
    dinv2_ref[...] = 1.0 / deg
    s0_ref[...] = t0_ref[...] * dinv


_scale_call = pl.pallas_call(
    _scale_body,
    out_shape=(jax.ShapeDtypeStruct((_NPAD, _C), _f32),
               jax.ShapeDtypeStruct((_NPAD, 1), _f32),
               jax.ShapeDtypeStruct((_NPAD, 1), _f32)),
)


def _combine_body(p_ref, s_ref, sa_ref, sb_ref, c_ref, o_ref):
    o_ref[...] = (sa_ref[...] * (p_ref[0] + p_ref[1] + s_ref[...])
                  + sb_ref[...] * c_ref[...])


_combine_call = pl.pallas_call(
    _combine_body,
    out_shape=jax.ShapeDtypeStruct((_NPAD, _C), _f32),
)


def _pool_body(t3_ref, batch_ref, bl_ref, o_ref):
    gids = lax.broadcasted_iota(_i32, (_G, _NPAD), 0)
    onehot = (gids == batch_ref[...]).astype(_f32)
    sums = jnp.dot(onehot, t3_ref[...], preferred_element_type=_f32, precision=lax.Precision.HIGHEST)
    counts = jnp.sum(onehot, axis=1, keepdims=True)
    o_ref[...] = sums / jnp.maximum(counts, 1.0) + bl_ref[...]


_pool_call = pl.pallas_call(
    _pool_body,
    out_shape=jax.ShapeDtypeStruct((_G, _C), _f32),
)


def kernel(x, edge_index, batch, W1, b1, W2, b2, W3, b3, Wl, bl):
    src = edge_index[0].astype(_i32)
    dst = edge_index[1].astype(_i32)
    epad = jnp.full((_EPAD - _E,), _DUMP, _i32)
    src2d = jnp.concatenate([src, epad]).reshape(_EPAD // _CHUNK, _CHUNK)
    dst2d = jnp.concatenate([dst, epad]).reshape(_EPAD // _CHUNK, _CHUNK)

    x_pad = jnp.pad(x.astype(_f32), ((0, _NPAD - _N), (0, 0)))
    t0, cmat = _prep_call(x_pad, W1, W2, W3, Wl,
                          b1.reshape(1, -1), b2.reshape(1, -1),
                          b3.reshape(1, -1))

    degp = _deg_call(dst2d).reshape(_NCORES, _NPAD, 1)
    s0, dinv, dinv2 = _scale_call(degp, t0)
    ones_col = jnp.ones((_NPAD, 1), _f32)

    p1 = _layer_call(s0, src2d, dst2d).reshape(_NCORES, _NPAD, _C)
    s1 = _combine_call(p1, s0, dinv2, dinv, cmat[0:1])
    p2 = _layer_call(s1, src2d, dst2d).reshape(_NCORES, _NPAD, _C)
    s2 = _combine_call(p2, s1, dinv2, dinv, cmat[1:2])
    p3 = _layer_call(s2, src2d, dst2d).reshape(_NCORES, _NPAD, _C)
    t3 = _combine_call(p3, s2, dinv, ones_col, cmat[2:3])

    batch_pad = jnp.concatenate(
        [batch.astype(_i32), jnp.full((_NPAD - _N,), _G, _i32)])
    out = _pool_call(t3, batch_pad.reshape(1, _NPAD), bl.reshape(1, -1))
    return out


# speedup vs baseline: 24.3484x; 1.0107x over previous
import functools

import jax
import jax.numpy as jnp
from jax import lax
from jax.experimental import pallas as pl
from jax.experimental.pallas import tpu as pltpu
from jax.experimental.pallas import tpu_sc as plsc

_N = 10000
_E = 320000
_C = 16
_G = 64

_NPAD = 10240
_DUMP = _N
_NCORES = 2
_NSUB = 16
_NTILES = _NCORES * _NSUB
_CHUNK = 128
_CPT = 80
_EPT = _CPT * _CHUNK
_EPAD = _NTILES * _EPT
_RPT = _NPAD // _NSUB

_f32 = jnp.float32
_i32 = jnp.int32

_MESH = plsc.VectorSubcoreMesh(core_axis_name="c", subcore_axis_name="s")


def _deg_body(dst_ref, out_ref, acc, idx, ones_v, zeros_v):
    core = lax.axis_index("c")
    sub = lax.axis_index("s")
    w = core * _NSUB + sub
    for i in range(_RPT // 16):
        zeros_v[pl.ds(i * 16, 16)] = jnp.zeros((16,), _f32)
    for i in range(_CHUNK // 16):
        ones_v[pl.ds(i * 16, 16)] = jnp.ones((16,), _f32)
    pltpu.sync_copy(zeros_v, acc.at[pl.ds(sub * _RPT, _RPT)])
    plsc.subcore_barrier()
    pltpu.sync_copy(dst_ref.at[pl.ds(w * _CPT, _CPT), :], idx)

    def body(j, carry):
        pltpu.sync_copy(ones_v, acc.at[idx.at[j]], add=True)
        return carry

    lax.fori_loop(0, _CPT, body, 0)
    plsc.subcore_barrier()
    pltpu.sync_copy(acc.at[pl.ds(sub * _RPT, _RPT)],
                    out_ref.at[pl.ds(core * _NPAD + sub * _RPT, _RPT)])


_deg_call = functools.partial(
    pl.kernel,
    out_type=jax.ShapeDtypeStruct((_NCORES * _NPAD,), _f32),
    mesh=_MESH,
    scratch_types=[
        pltpu.VMEM_SHARED((_NPAD,), _f32),
        pltpu.VMEM((_CPT, _CHUNK), _i32),
        pltpu.VMEM((_CHUNK,), _f32),
        pltpu.VMEM((_RPT,), _f32),
    ],
)(_deg_body)


_NBUF = 2


def _layer_body(s_ref, src_ref, dst_ref, out_ref, acc, sidx, didx, gbufs,
                zbuf, sems):
    core = lax.axis_index("c")
    sub = lax.axis_index("s")
    w = core * _NSUB + sub
    for i in range(_CHUNK):
        zbuf[i, :] = jnp.zeros((16,), _f32)
    for b in range(_RPT // _CHUNK):
        pltpu.sync_copy(zbuf, acc.at[pl.ds(sub * _RPT + b * _CHUNK, _CHUNK), :])
    for i in range(_NBUF):
        for q in range(_CHUNK // 16):
            sidx[_CPT + i, pl.ds(q * 16, 16)] = jnp.full((16,), _DUMP, _i32)
    plsc.subcore_barrier()
    pltpu.sync_copy(src_ref.at[pl.ds(w * _CPT, _CPT), :],
                    sidx.at[pl.ds(0, _CPT), :])
    pltpu.sync_copy(dst_ref.at[pl.ds(w * _CPT, _CPT), :], didx)

    for b in range(_NBUF):
        pltpu.async_copy(s_ref.at[sidx.at[b]], gbufs.at[b], sems.at[b])

    def body(jj, carry):
        c0 = jj * _NBUF
        for b in range(_NBUF):
            pltpu.make_async_copy(s_ref.at[sidx.at[c0 + b]], gbufs.at[b],
                                  sems.at[b]).wait()
            pltpu.sync_copy(gbufs.at[b], acc.at[didx.at[c0 + b]], add=True)
            pltpu.async_copy(s_ref.at[sidx.at[c0 + b + _NBUF]], gbufs.at[b],
                             sems.at[b])
        return carry

    lax.fori_loop(0, _CPT // _NBUF, body, 0)
    for b in range(_NBUF):
        pltpu.make_async_copy(s_ref.at[sidx.at[_CPT + b]], gbufs.at[b],
                              sems.at[b]).wait()
    plsc.subcore_barrier()
    pltpu.sync_copy(acc.at[pl.ds(sub * _RPT, _RPT), :],
                    out_ref.at[pl.ds(core * _NPAD + sub * _RPT, _RPT), :])


_layer_call = functools.partial(
    pl.kernel,
    out_type=jax.ShapeDtypeStruct((_NCORES * _NPAD, _C), _f32),
    mesh=_MESH,
    compiler_params=pltpu.CompilerParams(use_tc_tiling_on_sc=False),
    scratch_types=[
        pltpu.VMEM_SHARED((_NPAD, _C), _f32),
        pltpu.VMEM((_CPT + _NBUF, _CHUNK), _i32),
        pltpu.VMEM((_CPT, _CHUNK), _i32),
        pltpu.VMEM((_NBUF, _CHUNK, _C), _f32),
        pltpu.VMEM((_CHUNK, _C), _f32),
        pltpu.SemaphoreType.DMA((_NBUF,)),
    ],
)(_layer_body)


def _prep_body(x_ref, w1_ref, w2_ref, w3_ref, wl_ref, b1_ref, b2_ref,
               b3_ref, t0_ref, cmat_ref):
    f = _f32
    w3l = jnp.dot(w3_ref[...], wl_ref[...], preferred_element_type=f, precision=lax.Precision.HIGHEST)
    w23l = jnp.dot(w2_ref[...], w3l, preferred_element_type=f, precision=lax.Precision.HIGHEST)
    wc = jnp.dot(w1_ref[...], w23l, preferred_element_type=f, precision=lax.Precision.HIGHEST)
    t0_ref[...] = jnp.dot(x_ref[...], wc, preferred_element_type=f, precision=lax.Precision.HIGHEST)
    c1 = jnp.dot(b1_ref[...], w23l, preferred_element_type=f, precision=lax.Precision.HIGHEST)
    c2 = jnp.dot(b2_ref[...], w3l, preferred_element_type=f, precision=lax.Precision.HIGHEST)
    c3 = jnp.dot(b3_ref[...], wl_ref[...], preferred_element_type=f, precision=lax.Precision.HIGHEST)
    cmat_ref[...] = jnp.concatenate(
        [c1, c2, c3, jnp.zeros((5, _C), f)], axis=0)


_prep_call = pl.pallas_call(
    _prep_body,
    out_shape=(jax.ShapeDtypeStruct((_NPAD, _C), _f32),
               jax.ShapeDtypeStruct((8, _C), _f32)),
)


def _scale_body(p_ref, t0_ref, s0_ref, dinv_ref, dinv2_ref):
    deg = p_ref[0] + p_ref[1] + 1.0
    dinv = 1.0 / jnp.sqrt(deg)
    dinv_ref[...] = dinv
    dinv2_ref[...] = 1.0 / deg
    s0_ref[...] = t0_ref[...] * dinv


_scale_call = pl.pallas_call(
    _scale_body,
    out_shape=(jax.ShapeDtypeStruct((_NPAD, _C), _f32),
               jax.ShapeDtypeStruct((_NPAD, 1), _f32),
               jax.ShapeDtypeStruct((_NPAD, 1), _f32)),
)


def _combine_body(p_ref, s_ref, sa_ref, sb_ref, c_ref, o_ref):
    o_ref[...] = (sa_ref[...] * (p_ref[0] + p_ref[1] + s_ref[...])
                  + sb_ref[...] * c_ref[...])


_combine_call = pl.pallas_call(
    _combine_body,
    out_shape=jax.ShapeDtypeStruct((_NPAD, _C), _f32),
)


def _pool_body(t3_ref, batch_ref, bl_ref, o_ref):
    gids = lax.broadcasted_iota(_i32, (_G, _NPAD), 0)
    onehot = (gids == batch_ref[...]).astype(_f32)
    sums = jnp.dot(onehot, t3_ref[...], preferred_element_type=_f32, precision=lax.Precision.HIGHEST)
    counts = jnp.sum(onehot, axis=1, keepdims=True)
    o_ref[...] = sums / jnp.maximum(counts, 1.0) + bl_ref[...]


_pool_call = pl.pallas_call(
    _pool_body,
    out_shape=jax.ShapeDtypeStruct((_G, _C), _f32),
)


def kernel(x, edge_index, batch, W1, b1, W2, b2, W3, b3, Wl, bl):
    src = edge_index[0].astype(_i32)
    dst = edge_index[1].astype(_i32)
    epad = jnp.full((_EPAD - _E,), _DUMP, _i32)
    src2d = jnp.concatenate([src, epad]).reshape(_EPAD // _CHUNK, _CHUNK)
    dst2d = jnp.concatenate([dst, epad]).reshape(_EPAD // _CHUNK, _CHUNK)

    x_pad = jnp.pad(x.astype(_f32), ((0, _NPAD - _N), (0, 0)))
    t0, cmat = _prep_call(x_pad, W1, W2, W3, Wl,
                          b1.reshape(1, -1), b2.reshape(1, -1),
                          b3.reshape(1, -1))

    degp = _deg_call(dst2d).reshape(_NCORES, _NPAD, 1)
    s0, dinv, dinv2 = _scale_call(degp, t0)
    ones_col = jnp.ones((_NPAD, 1), _f32)

    p1 = _layer_call(s0, src2d, dst2d).reshape(_NCORES, _NPAD, _C)
    s1 = _combine_call(p1, s0, dinv2, dinv, cmat[0:1])
    p2 = _layer_call(s1, src2d, dst2d).reshape(_NCORES, _NPAD, _C)
    s2 = _combine_call(p2, s1, dinv2, dinv, cmat[1:2])
    p3 = _layer_call(s2, src2d, dst2d).reshape(_NCORES, _NPAD, _C)
    t3 = _combine_call(p3, s2, dinv, ones_col, cmat[2:3])

    batch_pad = jnp.concatenate(
        [batch.astype(_i32), jnp.full((_NPAD - _N,), _G, _i32)])
    out = _pool_call(t3, batch_pad.reshape(1, _NPAD), bl.reshape(1, -1))
    return out


# speedup vs baseline: 31.1205x; 1.2781x over previous
import functools

import jax
import jax.numpy as jnp
from jax import lax
from jax.experimental import pallas as pl
from jax.experimental.pallas import tpu as pltpu
from jax.experimental.pallas import tpu_sc as plsc

_N = 10000
_E = 320000
_C = 16
_G = 64

_NPAD = 10240
_DUMP = _N
_NCORES = 2
_NSUB = 16
_NTILES = _NCORES * _NSUB
_CHUNK = 128
_CPT = 80
_EPT = _CPT * _CHUNK
_EPAD = _NTILES * _EPT
_RPT = _NPAD // _NSUB

_f32 = jnp.float32
_i32 = jnp.int32

_MESH = plsc.VectorSubcoreMesh(core_axis_name="c", subcore_axis_name="s")


def _deg_body(dst_ref, out_ref, acc, idx, ones_v, zeros_v):
    core = lax.axis_index("c")
    sub = lax.axis_index("s")
    w = core * _NSUB + sub
    for i in range(_RPT // 16):
        zeros_v[pl.ds(i * 16, 16)] = jnp.zeros((16,), _f32)
    for i in range(_CHUNK // 16):
        ones_v[pl.ds(i * 16, 16)] = jnp.ones((16,), _f32)
    pltpu.sync_copy(zeros_v, acc.at[pl.ds(sub * _RPT, _RPT)])
    plsc.subcore_barrier()
    pltpu.sync_copy(dst_ref.at[pl.ds(w * _CPT, _CPT), :], idx)

    def body(j, carry):
        pltpu.sync_copy(ones_v, acc.at[idx.at[j]], add=True)
        return carry

    lax.fori_loop(0, _CPT, body, 0)
    plsc.subcore_barrier()
    pltpu.sync_copy(acc.at[pl.ds(sub * _RPT, _RPT)],
                    out_ref.at[pl.ds(core * _NPAD + sub * _RPT, _RPT)])


_deg_call = functools.partial(
    pl.kernel,
    out_type=jax.ShapeDtypeStruct((_NCORES * _NPAD,), _f32),
    mesh=_MESH,
    scratch_types=[
        pltpu.VMEM_SHARED((_NPAD,), _f32),
        pltpu.VMEM((_CPT, _CHUNK), _i32),
        pltpu.VMEM((_CHUNK,), _f32),
        pltpu.VMEM((_RPT,), _f32),
    ],
)(_deg_body)


_NBUF = 8


def _layer_body(s_ref, src_ref, dst_ref, out_ref, acc, sidx, didx, gbufs,
                zbuf, gsems, ssems):
    core = lax.axis_index("c")
    sub = lax.axis_index("s")
    w = core * _NSUB + sub
    for i in range(_CHUNK):
        zbuf[i, :] = jnp.zeros((16,), _f32)
    for b in range(_RPT // _CHUNK):
        pltpu.sync_copy(zbuf, acc.at[pl.ds(sub * _RPT + b * _CHUNK, _CHUNK), :])
    plsc.subcore_barrier()
    pltpu.sync_copy(src_ref.at[pl.ds(w * _CPT, _CPT), :], sidx)
    pltpu.sync_copy(dst_ref.at[pl.ds(w * _CPT, _CPT), :], didx)

    def body(jj, carry):
        c0 = jj * _NBUF
        for b in range(_NBUF):
            pltpu.async_copy(s_ref.at[sidx.at[c0 + b]], gbufs.at[b],
                             gsems.at[b])
        for b in range(_NBUF):
            pltpu.make_async_copy(s_ref.at[sidx.at[c0 + b]], gbufs.at[b],
                                  gsems.at[b]).wait()
            pltpu.async_copy(gbufs.at[b], acc.at[didx.at[c0 + b]],
                             ssems.at[b], add=True)
        for b in range(_NBUF):
            pltpu.make_async_copy(gbufs.at[b], acc.at[didx.at[c0 + b]],
                                  ssems.at[b]).wait()
        return carry

    lax.fori_loop(0, _CPT // _NBUF, body, 0)
    plsc.subcore_barrier()
    pltpu.sync_copy(acc.at[pl.ds(sub * _RPT, _RPT), :],
                    out_ref.at[pl.ds(core * _NPAD + sub * _RPT, _RPT), :])


_layer_call = functools.partial(
    pl.kernel,
    out_type=jax.ShapeDtypeStruct((_NCORES * _NPAD, _C), _f32),
    mesh=_MESH,
    compiler_params=pltpu.CompilerParams(use_tc_tiling_on_sc=False),
    scratch_types=[
        pltpu.VMEM_SHARED((_NPAD, _C), _f32),
        pltpu.VMEM((_CPT, _CHUNK), _i32),
        pltpu.VMEM((_CPT, _CHUNK), _i32),
        pltpu.VMEM((_NBUF, _CHUNK, _C), _f32),
        pltpu.VMEM((_CHUNK, _C), _f32),
        pltpu.SemaphoreType.DMA((_NBUF,)),
        pltpu.SemaphoreType.DMA((_NBUF,)),
    ],
)(_layer_body)


def _prep_body(x_ref, w1_ref, w2_ref, w3_ref, wl_ref, b1_ref, b2_ref,
               b3_ref, t0_ref, cmat_ref):
    f = _f32
    w3l = jnp.dot(w3_ref[...], wl_ref[...], preferred_element_type=f, precision=lax.Precision.HIGHEST)
    w23l = jnp.dot(w2_ref[...], w3l, preferred_element_type=f, precision=lax.Precision.HIGHEST)
    wc = jnp.dot(w1_ref[...], w23l, preferred_element_type=f, precision=lax.Precision.HIGHEST)
    t0_ref[...] = jnp.dot(x_ref[...], wc, preferred_element_type=f, precision=lax.Precision.HIGHEST)
    c1 = jnp.dot(b1_ref[...], w23l, preferred_element_type=f, precision=lax.Precision.HIGHEST)
    c2 = jnp.dot(b2_ref[...], w3l, preferred_element_type=f, precision=lax.Precision.HIGHEST)
    c3 = jnp.dot(b3_ref[...], wl_ref[...], preferred_element_type=f, precision=lax.Precision.HIGHEST)
    cmat_ref[...] = jnp.concatenate(
        [c1, c2, c3, jnp.zeros((5, _C), f)], axis=0)


_prep_call = pl.pallas_call(
    _prep_body,
    out_shape=(jax.ShapeDtypeStruct((_NPAD, _C), _f32),
               jax.ShapeDtypeStruct((8, _C), _f32)),
)


def _scale_body(p_ref, t0_ref, s0_ref, dinv_ref, dinv2_ref):
    deg = p_ref[0] + p_ref[1] + 1.0
    dinv = 1.0 / jnp.sqrt(deg)
    dinv_ref[...] = dinv
    dinv2_ref[...] = 1.0 / deg
    s0_ref[...] = t0_ref[...] * dinv


_scale_call = pl.pallas_call(
    _scale_body,
    out_shape=(jax.ShapeDtypeStruct((_NPAD, _C), _f32),
               jax.ShapeDtypeStruct((_NPAD, 1), _f32),
               jax.ShapeDtypeStruct((_NPAD, 1), _f32)),
)


def _combine_body(p_ref, s_ref, sa_ref, sb_ref, c_ref, o_ref):
    o_ref[...] = (sa_ref[...] * (p_ref[0] + p_ref[1] + s_ref[...])
                  + sb_ref[...] * c_ref[...])


_combine_call = pl.pallas_call(
    _combine_body,
    out_shape=jax.ShapeDtypeStruct((_NPAD, _C), _f32),
)


def _pool_body(t3_ref, batch_ref, bl_ref, o_ref):
    gids = lax.broadcasted_iota(_i32, (_G, _NPAD), 0)
    onehot = (gids == batch_ref[...]).astype(_f32)
    sums = jnp.dot(onehot, t3_ref[...], preferred_element_type=_f32, precision=lax.Precision.HIGHEST)
    counts = jnp.sum(onehot, axis=1, keepdims=True)
    o_ref[...] = sums / jnp.maximum(counts, 1.0) + bl_ref[...]


_pool_call = pl.pallas_call(
    _pool_body,
    out_shape=jax.ShapeDtypeStruct((_G, _C), _f32),
)


def kernel(x, edge_index, batch, W1, b1, W2, b2, W3, b3, Wl, bl):
    src = edge_index[0].astype(_i32)
    dst = edge_index[1].astype(_i32)
    epad = jnp.full((_EPAD - _E,), _DUMP, _i32)
    src2d = jnp.concatenate([src, epad]).reshape(_EPAD // _CHUNK, _CHUNK)
    dst2d = jnp.concatenate([dst, epad]).reshape(_EPAD // _CHUNK, _CHUNK)

    x_pad = jnp.pad(x.astype(_f32), ((0, _NPAD - _N), (0, 0)))
    t0, cmat = _prep_call(x_pad, W1, W2, W3, Wl,
                          b1.reshape(1, -1), b2.reshape(1, -1),
                          b3.reshape(1, -1))

    degp = _deg_call(dst2d).reshape(_NCORES, _NPAD, 1)
    s0, dinv, dinv2 = _scale_call(degp, t0)
    ones_col = jnp.ones((_NPAD, 1), _f32)

    p1 = _layer_call(s0, src2d, dst2d).reshape(_NCORES, _NPAD, _C)
    s1 = _combine_call(p1, s0, dinv2, dinv, cmat[0:1])
    p2 = _layer_call(s1, src2d, dst2d).reshape(_NCORES, _NPAD, _C)
    s2 = _combine_call(p2, s1, dinv2, dinv, cmat[1:2])
    p3 = _layer_call(s2, src2d, dst2d).reshape(_NCORES, _NPAD, _C)
    t3 = _combine_call(p3, s2, dinv, ones_col, cmat[2:3])

    batch_pad = jnp.concatenate(
        [batch.astype(_i32), jnp.full((_NPAD - _N,), _G, _i32)])
    out = _pool_call(t3, batch_pad.reshape(1, _NPAD), bl.reshape(1, -1))
    return out


# speedup vs baseline: 54.7638x; 1.7597x over previous
import functools

import jax
import jax.numpy as jnp
from jax import lax
from jax.experimental import pallas as pl
from jax.experimental.pallas import tpu as pltpu
from jax.experimental.pallas import tpu_sc as plsc

_N = 10000
_E = 320000
_C = 16
_G = 64

_NPAD = 10240
_DUMP = _N
_NCORES = 2
_NSUB = 16
_NTILES = _NCORES * _NSUB
_CHUNK = 128
_CPT = 80
_EPT = _CPT * _CHUNK
_EPAD = _NTILES * _EPT
_RPT = _NPAD // _NSUB

_f32 = jnp.float32
_i32 = jnp.int32

_MESH = plsc.VectorSubcoreMesh(core_axis_name="c", subcore_axis_name="s")


def _deg_body(dst_ref, out_ref, acc, idx, ones_v, zeros_v):
    core = lax.axis_index("c")
    sub = lax.axis_index("s")
    w = core * _NSUB + sub
    for i in range(_RPT // 16):
        zeros_v[pl.ds(i * 16, 16)] = jnp.zeros((16,), _f32)
    for i in range(_CHUNK // 16):
        ones_v[pl.ds(i * 16, 16)] = jnp.ones((16,), _f32)
    pltpu.sync_copy(zeros_v, acc.at[pl.ds(sub * _RPT, _RPT)])
    plsc.subcore_barrier()
    pltpu.sync_copy(dst_ref.at[pl.ds(w * _CPT, _CPT), :], idx)

    def body(j, carry):
        pltpu.sync_copy(ones_v, acc.at[idx.at[j]], add=True)
        return carry

    lax.fori_loop(0, _CPT, body, 0)
    plsc.subcore_barrier()
    pltpu.sync_copy(acc.at[pl.ds(sub * _RPT, _RPT)],
                    out_ref.at[pl.ds(core * _NPAD + sub * _RPT, _RPT)])


_deg_call = functools.partial(
    pl.kernel,
    out_type=jax.ShapeDtypeStruct((_NCORES * _NPAD,), _f32),
    mesh=_MESH,
    scratch_types=[
        pltpu.VMEM_SHARED((_NPAD,), _f32),
        pltpu.VMEM((_CPT, _CHUNK), _i32),
        pltpu.VMEM((_CHUNK,), _f32),
        pltpu.VMEM((_RPT,), _f32),
    ],
)(_deg_body)


_NBUF = 8


def _layer_body(s_ref, src_ref, dst_ref, out_ref, acc, sidx, didx, gbufs,
                zbuf, gsems, ssems):
    core = lax.axis_index("c")
    sub = lax.axis_index("s")
    w = core * _NSUB + sub
    for i in range(_CHUNK):
        zbuf[i, :] = jnp.zeros((16,), _f32)
    for b in range(_RPT // _CHUNK):
        pltpu.sync_copy(zbuf, acc.at[pl.ds(sub * _RPT + b * _CHUNK, _CHUNK), :])
    plsc.subcore_barrier()
    pltpu.sync_copy(src_ref.at[pl.ds(w * _CPT, _CPT), :], sidx)
    pltpu.sync_copy(dst_ref.at[pl.ds(w * _CPT, _CPT), :], didx)

    def body(jj, carry):
        c0 = jj * _NBUF
        for b in range(_NBUF):
            pltpu.async_copy(s_ref.at[sidx.at[c0 + b]], gbufs.at[b],
                             gsems.at[b])
        for b in range(_NBUF):
            pltpu.make_async_copy(s_ref.at[sidx.at[c0 + b]], gbufs.at[b],
                                  gsems.at[b]).wait()
            pltpu.async_copy(gbufs.at[b], acc.at[didx.at[c0 + b]],
                             ssems.at[b], add=True)
        for b in range(_NBUF):
            pltpu.make_async_copy(gbufs.at[b], acc.at[didx.at[c0 + b]],
                                  ssems.at[b]).wait()
        return carry

    lax.fori_loop(0, _CPT // _NBUF, body, 0)
    plsc.subcore_barrier()
    pltpu.sync_copy(acc.at[pl.ds(sub * _RPT, _RPT), :],
                    out_ref.at[pl.ds(core * _NPAD + sub * _RPT, _RPT), :])


_layer_call = functools.partial(
    pl.kernel,
    out_type=jax.ShapeDtypeStruct((_NCORES * _NPAD, _C), _f32),
    mesh=_MESH,
    compiler_params=pltpu.CompilerParams(use_tc_tiling_on_sc=False),
    scratch_types=[
        pltpu.VMEM_SHARED((_NPAD, _C), _f32),
        pltpu.VMEM((_CPT, _CHUNK), _i32),
        pltpu.VMEM((_CPT, _CHUNK), _i32),
        pltpu.VMEM((_NBUF, _CHUNK, _C), _f32),
        pltpu.VMEM((_CHUNK, _C), _f32),
        pltpu.SemaphoreType.DMA((_NBUF,)),
        pltpu.SemaphoreType.DMA((_NBUF,)),
    ],
)(_layer_body)


def _prep_body(x_ref, w1_ref, w2_ref, w3_ref, wl_ref, b1_ref, b2_ref,
               b3_ref, t0_ref, cmat_ref):
    f = _f32
    w3l = jnp.dot(w3_ref[...], wl_ref[...], preferred_element_type=f, precision=lax.Precision.HIGHEST)
    w23l = jnp.dot(w2_ref[...], w3l, preferred_element_type=f, precision=lax.Precision.HIGHEST)
    wc = jnp.dot(w1_ref[...], w23l, preferred_element_type=f, precision=lax.Precision.HIGHEST)
    t0_ref[...] = jnp.dot(x_ref[...], wc, preferred_element_type=f, precision=lax.Precision.HIGHEST)
    c1 = jnp.dot(b1_ref[...], w23l, preferred_element_type=f, precision=lax.Precision.HIGHEST)
    c2 = jnp.dot(b2_ref[...], w3l, preferred_element_type=f, precision=lax.Precision.HIGHEST)
    c3 = jnp.dot(b3_ref[...], wl_ref[...], preferred_element_type=f, precision=lax.Precision.HIGHEST)
    cmat_ref[...] = jnp.concatenate(
        [c1, c2, c3, jnp.zeros((5, _C), f)], axis=0)


_prep_call = pl.pallas_call(
    _prep_body,
    out_shape=(jax.ShapeDtypeStruct((_NPAD, _C), _f32),
               jax.ShapeDtypeStruct((8, _C), _f32)),
)


def _scale_body(p_ref, t0_ref, s0_ref, dinv_ref, dinv2_ref):
    deg = p_ref[0] + p_ref[1] + 1.0
    dinv = 1.0 / jnp.sqrt(deg)
    dinv_ref[...] = dinv
    dinv2_ref[...] = 1.0 / deg
    s0_ref[...] = t0_ref[...] * dinv


_scale_call = pl.pallas_call(
    _scale_body,
    out_shape=(jax.ShapeDtypeStruct((_NPAD, _C), _f32),
               jax.ShapeDtypeStruct((_NPAD, 1), _f32),
               jax.ShapeDtypeStruct((_NPAD, 1), _f32)),
)


def _combine_body(p_ref, s_ref, sa_ref, sb_ref, c_ref, o_ref):
    o_ref[...] = (sa_ref[...] * (p_ref[0] + p_ref[1] + s_ref[...])
                  + sb_ref[...] * c_ref[...])


_combine_call = pl.pallas_call(
    _combine_body,
    out_shape=jax.ShapeDtypeStruct((_NPAD, _C), _f32),
)


def _pool_body(p_ref, s_ref, dinv_ref, c_ref, batch_ref, bl_ref, o_ref):
    t3 = (dinv_ref[...] * (p_ref[0] + p_ref[1] + s_ref[...]) + c_ref[...])
    gids = lax.broadcasted_iota(_i32, (_G, _NPAD), 0)
    onehot = (gids == batch_ref[...]).astype(_f32)
    sums = jnp.dot(onehot, t3, preferred_element_type=_f32, precision=lax.Precision.HIGHEST)
    counts = jnp.sum(onehot, axis=1, keepdims=True)
    o_ref[...] = sums / jnp.maximum(counts, 1.0) + bl_ref[...]


_pool_call = pl.pallas_call(
    _pool_body,
    out_shape=jax.ShapeDtypeStruct((_G, _C), _f32),
)


def kernel(x, edge_index, batch, W1, b1, W2, b2, W3, b3, Wl, bl):
    src = edge_index[0].astype(_i32)
    dst = edge_index[1].astype(_i32)
    epad = _DUMP + jnp.arange(_EPAD - _E, dtype=_i32) % (_NPAD - _N)
    src2d = jnp.concatenate([src, epad]).reshape(_EPAD // _CHUNK, _CHUNK)
    dst2d = jnp.concatenate([dst, epad]).reshape(_EPAD // _CHUNK, _CHUNK)

    x_pad = jnp.pad(x.astype(_f32), ((0, _NPAD - _N), (0, 0)))
    t0, cmat = _prep_call(x_pad, W1, W2, W3, Wl,
                          b1.reshape(1, -1), b2.reshape(1, -1),
                          b3.reshape(1, -1))

    degp = _deg_call(dst2d).reshape(_NCORES, _NPAD, 1)
    s0, dinv, dinv2 = _scale_call(degp, t0)

    p1 = _layer_call(s0, src2d, dst2d).reshape(_NCORES, _NPAD, _C)
    s1 = _combine_call(p1, s0, dinv2, dinv, cmat[0:1])
    p2 = _layer_call(s1, src2d, dst2d).reshape(_NCORES, _NPAD, _C)
    s2 = _combine_call(p2, s1, dinv2, dinv, cmat[1:2])
    p3 = _layer_call(s2, src2d, dst2d).reshape(_NCORES, _NPAD, _C)

    batch_pad = jnp.concatenate(
        [batch.astype(_i32), jnp.full((_NPAD - _N,), _G, _i32)])
    out = _pool_call(p3, s2, dinv, cmat[2:3], batch_pad.reshape(1, _NPAD),
                     bl.reshape(1, -1))
    return out
